# baseline (speedup 1.0000x reference)
import jax
import jax.numpy as jnp
from jax.experimental import pallas as pl
from jax.experimental.pallas import tpu as pltpu


_NCHUNK = 16


def _copy_body(src_ref, dst_ref, sems):
    rows = src_ref.shape[0]
    ch = rows // _NCHUNK
    copies = [
        pltpu.make_async_copy(
            src_ref.at[pl.ds(i * ch, ch)],
            dst_ref.at[pl.ds(i * ch, ch)],
            sems.at[i],
        )
        for i in range(_NCHUNK)
    ]
    for c in copies:
        c.start()
    for c in copies:
        c.wait()


def kernel(x, table):
    del x
    out = pl.pallas_call(
        _copy_body,
        in_specs=[pl.BlockSpec(memory_space=pl.ANY)],
        out_specs=pl.BlockSpec(memory_space=pl.ANY),
        out_shape=jax.ShapeDtypeStruct(table.shape, table.dtype),
        scratch_shapes=[pltpu.SemaphoreType.DMA((_NCHUNK,))],
    )(table)
    return out[None]


# speedup vs baseline: 36.9275x; 36.9275x over previous
import jax
import jax.numpy as jnp
from jax.experimental import pallas as pl
from jax.experimental.pallas import tpu as pltpu


_BLOCK_ROWS = 512


def _copy_body(src_ref, dst_ref):
    dst_ref[...] = src_ref[...]


def kernel(x, table):
    del x
    rows, dim = table.shape
    grid = (rows // _BLOCK_ROWS,)
    out = pl.pallas_call(
        _copy_body,
        grid=grid,
        in_specs=[pl.BlockSpec((_BLOCK_ROWS, dim), lambda i: (i, 0))],
        out_specs=pl.BlockSpec((_BLOCK_ROWS, dim), lambda i: (i, 0)),
        out_shape=jax.ShapeDtypeStruct(table.shape, table.dtype),
    )(table)
    return out[None]


# speedup vs baseline: 43.0066x; 1.1646x over previous
import jax
import jax.numpy as jnp
from jax.experimental import pallas as pl
from jax.experimental.pallas import tpu as pltpu


_BLOCK_ROWS = 1024


def _copy_body(src_ref, dst_ref):
    dst_ref[...] = src_ref[...]


def kernel(x, table):
    del x
    rows, dim = table.shape
    grid = (rows // _BLOCK_ROWS,)
    out = pl.pallas_call(
        _copy_body,
        grid=grid,
        in_specs=[pl.BlockSpec((_BLOCK_ROWS, dim), lambda i: (i, 0))],
        out_specs=pl.BlockSpec((_BLOCK_ROWS, dim), lambda i: (i, 0)),
        out_shape=jax.ShapeDtypeStruct(table.shape, table.dtype),
    )(table)
    return out[None]


# speedup vs baseline: 46.2939x; 1.0764x over previous
import jax
import jax.numpy as jnp
from jax.experimental import pallas as pl
from jax.experimental.pallas import tpu as pltpu


_BLOCK_ROWS = 2048


def _copy_body(src_ref, dst_ref):
    dst_ref[...] = src_ref[...]


def kernel(x, table):
    del x
    rows, dim = table.shape
    grid = (rows // _BLOCK_ROWS,)
    out = pl.pallas_call(
        _copy_body,
        grid=grid,
        in_specs=[pl.BlockSpec((_BLOCK_ROWS, dim), lambda i: (i, 0))],
        out_specs=pl.BlockSpec((_BLOCK_ROWS, dim), lambda i: (i, 0)),
        out_shape=jax.ShapeDtypeStruct(table.shape, table.dtype),
    )(table)
    return out[None]


# speedup vs baseline: 49.1820x; 1.0624x over previous
import jax
import jax.numpy as jnp
from jax.experimental import pallas as pl
from jax.experimental.pallas import tpu as pltpu


_BLOCK_ROWS = 4096


def _copy_body(src_ref, dst_ref):
    dst_ref[...] = src_ref[...]


def kernel(x, table):
    del x
    rows, dim = table.shape
    grid = (rows // _BLOCK_ROWS,)
    out = pl.pallas_call(
        _copy_body,
        grid=grid,
        in_specs=[pl.BlockSpec((_BLOCK_ROWS, dim), lambda i: (i, 0))],
        out_specs=pl.BlockSpec((_BLOCK_ROWS, dim), lambda i: (i, 0)),
        out_shape=jax.ShapeDtypeStruct(table.shape, table.dtype),
    )(table)
    return out[None]
